# baseline (speedup 1.0000x reference)
import functools

import jax
import jax.numpy as jnp
from jax import lax
from jax.experimental import pallas as pl
from jax.experimental.pallas import tpu as pltpu
from jax.experimental.pallas import tpu_sc as plsc

N = 10000
E = 160000
D = 128
DE = 16
H = 128
TWO_N = 2 * N

NC = 2
NS = 16
NW = NC * NS

BN = 1000
NB_NODE = N // BN
BE = 1000
NB_EDGE = E // BE
CH = 128
NCHUNK = E // CH
KMAX = -(-NCHUNK // NW)

@functools.cache
def _mesh():
    return plsc.VectorSubcoreMesh(core_axis_name="c", subcore_axis_name="s",
                                  num_cores=NC, num_subcores=NS)


def _table_body(x_ref, nt_ref, w_ref, b_ref, o_ref):
    xb = x_ref[...]
    nt = nt_ref[...]
    p0 = jnp.dot(xb, w_ref[0, 0, 0], preferred_element_type=jnp.float32) + b_ref[0, 0, 0, 0]
    p1 = jnp.dot(xb, w_ref[0, 1, 0], preferred_element_type=jnp.float32) + b_ref[0, 1, 0, 0]
    o_ref[0, 0] = jnp.where(nt == 0, p0, p1)


def _build_table(x0, nt2, wstk, bstk):
    return pl.pallas_call(
        _table_body,
        grid=(2, 2, NB_NODE),
        in_specs=[
            pl.BlockSpec((BN, D), lambda h, o, nb: (nb, 0)),
            pl.BlockSpec((BN, 1), lambda h, o, nb: (nb, 0)),
            pl.BlockSpec((1, 2, 1, D, D), lambda h, o, nb: (h, 0, o, 0, 0)),
            pl.BlockSpec((1, 2, 1, 1, D), lambda h, o, nb: (h, 0, o, 0, 0)),
        ],
        out_specs=pl.BlockSpec((1, 1, BN, D), lambda h, o, nb: (h, o, nb, 0)),
        out_shape=jax.ShapeDtypeStruct((2, 2, N, D), jnp.float32),
    )(x0, nt2, wstk, bstk[:, :, :, None, :])


def _gather_body(nchunk, t_hbm, row_hbm, col_hbm, nt_hbm, g_hbm, p_hbm,
                 ntbuf, rbuf, cbuf, idxbuf, pbuf, dstbuf,
                 si0, si1, sg10, sg11, sg20, sg21, so0, so1):
    kmax = -(-nchunk // NW)
    c = lax.axis_index("c")
    s = lax.axis_index("s")
    w = s * NC + c
    pltpu.sync_copy(nt_hbm, ntbuf)
    sin = (si0, si1)
    sg1 = (sg10, sg11)
    sg2 = (sg20, sg21)
    sout = (so0, so1)

    def cid_of(j):
        return w + NW * j

    def valid(j):
        return (j >= 0) & (cid_of(j) < nchunk)

    def start_in(j, slot):
        @pl.when(valid(j))
        def _():
            eb = cid_of(j) * CH
            pltpu.async_copy(row_hbm.at[pl.ds(eb, CH)], rbuf.at[slot], sin[slot])
            pltpu.async_copy(col_hbm.at[pl.ds(eb, CH)], cbuf.at[slot], sin[slot])

    def step(j, slot):
        oslot = 1 - slot
        start_in(j + 1, oslot)

        @pl.when(valid(j - 2))
        def _():
            eb2 = cid_of(j - 2) * CH
            pltpu.make_async_copy(dstbuf.at[slot], g_hbm.at[pl.ds(eb2, CH)], sout[slot]).wait()
            pltpu.make_async_copy(pbuf.at[slot], p_hbm.at[pl.ds(eb2, CH)], sout[slot]).wait()

        @pl.when(valid(j))
        def _():
            eb = cid_of(j) * CH
            pltpu.make_async_copy(row_hbm.at[pl.ds(eb, CH)], rbuf.at[slot], sin[slot]).wait()
            pltpu.make_async_copy(col_hbm.at[pl.ds(eb, CH)], cbuf.at[slot], sin[slot]).wait()
            for g in range(8):
                sl = pl.ds(g * 16, 16)
                r16 = rbuf[slot, sl]
                c16 = cbuf[slot, sl]
                tr = plsc.load_gather(ntbuf, [r16])
                tc = plsc.load_gather(ntbuf, [c16])
                idxbuf[slot, 0, sl] = tc * N + r16
                idxbuf[slot, 1, sl] = TWO_N + tr * N + c16
                pbuf[slot, sl] = 2 * tr + tc

        @pl.when(valid(j))
        def _():
            pltpu.async_copy(t_hbm.at[idxbuf.at[slot, 0]], dstbuf.at[slot], sg1[slot])

        @pl.when(valid(j - 1))
        def _():
            eb1 = cid_of(j - 1) * CH
            pltpu.make_async_copy(t_hbm.at[idxbuf.at[oslot, 1]], dstbuf.at[oslot], sg2[oslot]).wait()
            pltpu.async_copy(dstbuf.at[oslot], g_hbm.at[pl.ds(eb1, CH)], sout[oslot])
            pltpu.async_copy(pbuf.at[oslot], p_hbm.at[pl.ds(eb1, CH)], sout[oslot])

        @pl.when(valid(j))
        def _():
            pltpu.make_async_copy(t_hbm.at[idxbuf.at[slot, 0]], dstbuf.at[slot], sg1[slot]).wait()
            pltpu.async_copy(t_hbm.at[idxbuf.at[slot, 1]], dstbuf.at[slot], sg2[slot], add=True)

    start_in(0, 0)
    nit = kmax + 2 + (kmax % 2)

    def two(k2, carry):
        step(2 * k2, 0)
        step(2 * k2 + 1, 1)
        return carry

    lax.fori_loop(0, nit // 2, two, 0)


@functools.cache
def _sc_gather_kernel(n_edges):
    return pl.kernel(
        functools.partial(_gather_body, n_edges // CH),
        out_type=(
            jax.ShapeDtypeStruct((n_edges, D), jnp.float32),
            jax.ShapeDtypeStruct((n_edges,), jnp.int32),
        ),
        mesh=_mesh(),
        scratch_types=[
            pltpu.VMEM((N,), jnp.int32),
            pltpu.VMEM((2, CH), jnp.int32),
            pltpu.VMEM((2, CH), jnp.int32),
            pltpu.VMEM((2, 2, CH), jnp.int32),
            pltpu.VMEM((2, CH), jnp.int32),
            pltpu.VMEM((2, CH, D), jnp.float32),
            pltpu.SemaphoreType.DMA,
            pltpu.SemaphoreType.DMA,
            pltpu.SemaphoreType.DMA,
            pltpu.SemaphoreType.DMA,
            pltpu.SemaphoreType.DMA,
            pltpu.SemaphoreType.DMA,
            pltpu.SemaphoreType.DMA,
            pltpu.SemaphoreType.DMA,
        ],
        compiler_params=pltpu.CompilerParams(needs_layout_passes=False),
    )


def _sc_gather(table, row, col, nt):
    return _sc_gather_kernel(row.shape[0])(table, row, col, nt)


def _edge_body(g_ref, ea_ref, p_ref, w1a_ref, w2_ref, b2_ref, o_ref):
    gb = g_ref[...]
    ea = ea_ref[...]
    pb = p_ref[...]
    colj = lax.broadcasted_iota(jnp.int32, (BE, 4 * DE), 1) // DE
    ea4 = jnp.concatenate([ea, ea, ea, ea], axis=1)
    a64 = jnp.where(colj == pb, ea4, 0.0)
    pre = gb + jnp.dot(a64, w1a_ref[...], preferred_element_type=jnp.float32)
    h1 = jnp.maximum(pre, 0.0)
    acc = jnp.zeros((BE, H), jnp.float32)
    b2s = jnp.zeros((BE, H), jnp.float32)
    for j in range(4):
        hj = jnp.where(pb == j, h1, 0.0)
        acc = acc + jnp.dot(hj, w2_ref[j], preferred_element_type=jnp.float32)
        b2s = b2s + jnp.where(pb == j, b2_ref[j], 0.0)
    o_ref[...] = jnp.maximum(acc + b2s, 0.0)


def _edge_mlp(g, ea, p2, w1a, w2s, b2s):
    n = g.shape[0]
    return pl.pallas_call(
        _edge_body,
        grid=(n // BE,),
        in_specs=[
            pl.BlockSpec((BE, H), lambda i: (i, 0)),
            pl.BlockSpec((BE, DE), lambda i: (i, 0)),
            pl.BlockSpec((BE, 1), lambda i: (i, 0)),
            pl.BlockSpec((4 * DE, H), lambda i: (0, 0)),
            pl.BlockSpec((4, H, H), lambda i: (0, 0, 0)),
            pl.BlockSpec((4, H), lambda i: (0, 0)),
        ],
        out_specs=pl.BlockSpec((BE, H), lambda i: (i, 0)),
        out_shape=jax.ShapeDtypeStruct((n, H), jnp.float32),
    )(g, ea, p2, w1a, w2s, b2s)


def _scatter_body(nchunk, eo_hbm, row_hbm, z_hbm, agg_hbm, shared, vbuf, idxbuf,
                  si0, si1, ss0, ss1):
    kmax = -(-nchunk // NW)
    c = lax.axis_index("c")
    s = lax.axis_index("s")
    r0 = 632
    lo = s * r0

    @pl.when(s < NS - 1)
    def _():
        pltpu.sync_copy(z_hbm.at[pl.ds(lo, r0)], shared.at[pl.ds(lo, r0)])

    @pl.when(s == NS - 1)
    def _():
        pltpu.sync_copy(z_hbm.at[pl.ds(lo, N - (NS - 1) * r0)],
                        shared.at[pl.ds(lo, N - (NS - 1) * r0)])

    plsc.subcore_barrier()

    sin = (si0, si1)
    ssc = (ss0, ss1)

    def cid_of(j):
        return c + NC * s + NW * j

    def valid(j):
        return (j >= 0) & (cid_of(j) < nchunk)

    def start_in(j, slot):
        @pl.when(valid(j))
        def _():
            eb = cid_of(j) * CH
            pltpu.async_copy(row_hbm.at[pl.ds(eb, CH)], idxbuf.at[slot], sin[slot])
            pltpu.async_copy(eo_hbm.at[pl.ds(eb, CH)], vbuf.at[slot], sin[slot])

    def step(j, slot):
        oslot = 1 - slot

        @pl.when(valid(j - 1))
        def _():
            pltpu.make_async_copy(vbuf.at[oslot], shared.at[idxbuf.at[oslot]], ssc[oslot]).wait()

        start_in(j + 1, oslot)

        @pl.when(valid(j))
        def _():
            eb = cid_of(j) * CH
            pltpu.make_async_copy(row_hbm.at[pl.ds(eb, CH)], idxbuf.at[slot], sin[slot]).wait()
            pltpu.make_async_copy(eo_hbm.at[pl.ds(eb, CH)], vbuf.at[slot], sin[slot]).wait()
            pltpu.async_copy(vbuf.at[slot], shared.at[idxbuf.at[slot]], ssc[slot], add=True)

    start_in(0, 0)
    nit = kmax + 2 + (kmax % 2)

    def two(k2, carry):
        step(2 * k2, 0)
        step(2 * k2 + 1, 1)
        return carry

    lax.fori_loop(0, nit // 2, two, 0)
    plsc.subcore_barrier()

    @pl.when(s < NS - 1)
    def _():
        pltpu.sync_copy(shared.at[pl.ds(lo, r0)], agg_hbm.at[c, pl.ds(lo, r0)])

    @pl.when(s == NS - 1)
    def _():
        pltpu.sync_copy(shared.at[pl.ds(lo, N - (NS - 1) * r0)],
                        agg_hbm.at[c, pl.ds(lo, N - (NS - 1) * r0)])


@functools.cache
def _sc_scatter_kernel(n_edges):
    return pl.kernel(
        functools.partial(_scatter_body, n_edges // CH),
        out_type=jax.ShapeDtypeStruct((2, N, D), jnp.float32),
        mesh=_mesh(),
        scratch_types=[
            pltpu.VMEM_SHARED((N, D), jnp.float32),
            pltpu.VMEM((2, CH, D), jnp.float32),
            pltpu.VMEM((2, CH), jnp.int32),
            pltpu.SemaphoreType.DMA,
            pltpu.SemaphoreType.DMA,
            pltpu.SemaphoreType.DMA,
            pltpu.SemaphoreType.DMA,
        ],
        compiler_params=pltpu.CompilerParams(needs_layout_passes=False),
    )


def _sc_scatter(e_out, row, zeros):
    return _sc_scatter_kernel(row.shape[0])(e_out, row, zeros)


def _node_body(x_ref, a_ref, b_ref, nt_ref, w1_ref, b1_ref, w2_ref, b2_ref, o_ref):
    xb = x_ref[...]
    ag = (a_ref[0] + a_ref[1]) + (b_ref[0] + b_ref[1])
    nt = nt_ref[...]
    outs = []
    for t in range(2):
        h1 = jnp.maximum(
            jnp.dot(xb, w1_ref[t, :D], preferred_element_type=jnp.float32)
            + jnp.dot(ag, w1_ref[t, D:], preferred_element_type=jnp.float32)
            + b1_ref[t], 0.0)
        outs.append(jnp.dot(h1, w2_ref[t], preferred_element_type=jnp.float32) + b2_ref[t])
    o_ref[...] = jnp.where(nt == 0, outs[0], outs[1]) + xb


def _node_mlp(x0, aggp, aggq, nt2, w1, b1, w2, b2):
    return pl.pallas_call(
        _node_body,
        grid=(NB_NODE,),
        in_specs=[
            pl.BlockSpec((BN, D), lambda i: (i, 0)),
            pl.BlockSpec((2, BN, D), lambda i: (0, i, 0)),
            pl.BlockSpec((2, BN, D), lambda i: (0, i, 0)),
            pl.BlockSpec((BN, 1), lambda i: (i, 0)),
            pl.BlockSpec((2, 2 * D, H), lambda i: (0, 0, 0)),
            pl.BlockSpec((2, H), lambda i: (0, 0)),
            pl.BlockSpec((2, H, H), lambda i: (0, 0, 0)),
            pl.BlockSpec((2, H), lambda i: (0, 0)),
        ],
        out_specs=pl.BlockSpec((BN, H), lambda i: (i, 0)),
        out_shape=jax.ShapeDtypeStruct((N, H), jnp.float32),
    )(x0, aggp, aggq, nt2, w1, b1, w2, b2)


def kernel(x, edge_attr, params, edge_index, node_type):
    x0 = x[:, 0]
    row = edge_index[0]
    col = edge_index[1]
    nt = node_type.astype(jnp.int32)
    nt2 = nt[:, None]

    w1 = {(a, b): params["edge_%d_%d_W1" % (a, b)] for a in (0, 1) for b in (0, 1)}
    b1 = {(a, b): params["edge_%d_%d_b1" % (a, b)] for a in (0, 1) for b in (0, 1)}
    wstk = jnp.stack([
        jnp.stack([jnp.stack([w1[(own, other)][:D] for other in (0, 1)])
                   for own in (0, 1)]),
        jnp.stack([jnp.stack([w1[(other, own)][D:2 * D] for other in (0, 1)])
                   for own in (0, 1)]),
    ])
    bstk = jnp.stack([
        jnp.stack([jnp.stack([b1[(own, other)] for other in (0, 1)])
                   for own in (0, 1)]),
        jnp.zeros((2, 2, H), jnp.float32),
    ])

    w1a = jnp.concatenate([w1[(j // 2, j % 2)][2 * D:] for j in range(4)], axis=0)
    w2s = jnp.stack([params["edge_%d_%d_W2" % (j // 2, j % 2)] for j in range(4)])
    b2s = jnp.stack([params["edge_%d_%d_b2" % (j // 2, j % 2)] for j in range(4)])

    wn1 = jnp.stack([params["node_%d_W1" % t] for t in (0, 1)])
    bn1 = jnp.stack([params["node_%d_b1" % t] for t in (0, 1)])
    wn2 = jnp.stack([params["node_%d_W2" % t] for t in (0, 1)])
    bn2 = jnp.stack([params["node_%d_b2" % t] for t in (0, 1)])

    table = _build_table(x0, nt2, wstk, bstk).reshape(4 * N, D)
    e2 = E // 2
    z = jnp.zeros((N, D), jnp.float32)
    halves = []
    for lo in (0, e2):
        rr = lax.slice(row, (lo,), (lo + e2,))
        cc = lax.slice(col, (lo,), (lo + e2,))
        ea = lax.slice(edge_attr, (lo, 0), (lo + e2, DE))
        g2, p = _sc_gather(table, rr, cc, nt)
        eo = _edge_mlp(g2, ea, p[:, None], w1a, w2s, b2s)
        halves.append((eo, _sc_scatter(eo, rr, z)))
    e_out = jnp.concatenate([halves[0][0], halves[1][0]], axis=0)
    h = _node_mlp(x0, halves[0][1], halves[1][1], nt2, wn1, bn1, wn2, bn2)
    return (h[:, None, :], edge_index, e_out, node_type)


# speedup vs baseline: 1.0102x; 1.0102x over previous
import functools

import jax
import jax.numpy as jnp
from jax import lax
from jax.experimental import pallas as pl
from jax.experimental.pallas import tpu as pltpu
from jax.experimental.pallas import tpu_sc as plsc

N = 10000
E = 160000
D = 128
DE = 16
H = 128
TWO_N = 2 * N

NC = 2
NS = 16
NW = NC * NS

BN = 1000
NB_NODE = N // BN
BE = 1000
NB_EDGE = E // BE
CH = 128
NCHUNK = E // CH
KMAX = -(-NCHUNK // NW)

@functools.cache
def _mesh():
    return plsc.VectorSubcoreMesh(core_axis_name="c", subcore_axis_name="s",
                                  num_cores=NC, num_subcores=NS)


def _table_body(x_ref, nt_ref, w_ref, b_ref, o_ref):
    xb = x_ref[...]
    nt = nt_ref[...]
    p0 = jnp.dot(xb, w_ref[0, 0, 0], preferred_element_type=jnp.float32) + b_ref[0, 0, 0, 0]
    p1 = jnp.dot(xb, w_ref[0, 1, 0], preferred_element_type=jnp.float32) + b_ref[0, 1, 0, 0]
    o_ref[0, 0] = jnp.where(nt == 0, p0, p1)


def _build_table(x0, nt2, wstk, bstk):
    return pl.pallas_call(
        _table_body,
        grid=(2, 2, NB_NODE),
        in_specs=[
            pl.BlockSpec((BN, D), lambda h, o, nb: (nb, 0)),
            pl.BlockSpec((BN, 1), lambda h, o, nb: (nb, 0)),
            pl.BlockSpec((1, 2, 1, D, D), lambda h, o, nb: (h, 0, o, 0, 0)),
            pl.BlockSpec((1, 2, 1, 1, D), lambda h, o, nb: (h, 0, o, 0, 0)),
        ],
        out_specs=pl.BlockSpec((1, 1, BN, D), lambda h, o, nb: (h, o, nb, 0)),
        out_shape=jax.ShapeDtypeStruct((2, 2, N, D), jnp.float32),
    )(x0, nt2, wstk, bstk[:, :, :, None, :])


def _gather_body(nchunk, t_hbm, row_hbm, col_hbm, nt_hbm, g_hbm, p_hbm,
                 ntbuf, rbuf, cbuf, idxbuf, pbuf, dstbuf,
                 si0, si1, sg10, sg11, sg20, sg21, so0, so1):
    kmax = -(-nchunk // NW)
    c = lax.axis_index("c")
    s = lax.axis_index("s")
    w = s * NC + c
    pltpu.sync_copy(nt_hbm, ntbuf)
    sin = (si0, si1)
    sg1 = (sg10, sg11)
    sg2 = (sg20, sg21)
    sout = (so0, so1)

    def cid_of(j):
        return w + NW * j

    def valid(j):
        return (j >= 0) & (cid_of(j) < nchunk)

    def start_in(j, slot):
        @pl.when(valid(j))
        def _():
            eb = cid_of(j) * CH
            pltpu.async_copy(row_hbm.at[pl.ds(eb, CH)], rbuf.at[slot], sin[slot])
            pltpu.async_copy(col_hbm.at[pl.ds(eb, CH)], cbuf.at[slot], sin[slot])

    def step(j, slot):
        oslot = 1 - slot
        start_in(j + 1, oslot)

        @pl.when(valid(j - 2))
        def _():
            eb2 = cid_of(j - 2) * CH
            pltpu.make_async_copy(dstbuf.at[slot], g_hbm.at[pl.ds(eb2, CH)], sout[slot]).wait()
            pltpu.make_async_copy(pbuf.at[slot], p_hbm.at[pl.ds(eb2, CH)], sout[slot]).wait()

        @pl.when(valid(j))
        def _():
            eb = cid_of(j) * CH
            pltpu.make_async_copy(row_hbm.at[pl.ds(eb, CH)], rbuf.at[slot], sin[slot]).wait()
            pltpu.make_async_copy(col_hbm.at[pl.ds(eb, CH)], cbuf.at[slot], sin[slot]).wait()
            for g in range(8):
                sl = pl.ds(g * 16, 16)
                r16 = rbuf[slot, sl]
                c16 = cbuf[slot, sl]
                tr = plsc.load_gather(ntbuf, [r16])
                tc = plsc.load_gather(ntbuf, [c16])
                idxbuf[slot, 0, sl] = tc * N + r16
                idxbuf[slot, 1, sl] = TWO_N + tr * N + c16
                pbuf[slot, sl] = 2 * tr + tc

        @pl.when(valid(j))
        def _():
            pltpu.async_copy(t_hbm.at[idxbuf.at[slot, 0]], dstbuf.at[slot], sg1[slot])

        @pl.when(valid(j - 1))
        def _():
            eb1 = cid_of(j - 1) * CH
            pltpu.make_async_copy(t_hbm.at[idxbuf.at[oslot, 1]], dstbuf.at[oslot], sg2[oslot]).wait()
            pltpu.async_copy(dstbuf.at[oslot], g_hbm.at[pl.ds(eb1, CH)], sout[oslot])
            pltpu.async_copy(pbuf.at[oslot], p_hbm.at[pl.ds(eb1, CH)], sout[oslot])

        @pl.when(valid(j))
        def _():
            pltpu.make_async_copy(t_hbm.at[idxbuf.at[slot, 0]], dstbuf.at[slot], sg1[slot]).wait()
            pltpu.async_copy(t_hbm.at[idxbuf.at[slot, 1]], dstbuf.at[slot], sg2[slot], add=True)

    start_in(0, 0)
    nit = kmax + 2 + (kmax % 2)

    def two(k2, carry):
        step(2 * k2, 0)
        step(2 * k2 + 1, 1)
        return carry

    lax.fori_loop(0, nit // 2, two, 0)


@functools.cache
def _sc_gather_kernel(n_edges):
    return pl.kernel(
        functools.partial(_gather_body, n_edges // CH),
        out_type=(
            jax.ShapeDtypeStruct((n_edges, D), jnp.float32),
            jax.ShapeDtypeStruct((n_edges,), jnp.int32),
        ),
        mesh=_mesh(),
        scratch_types=[
            pltpu.VMEM((N,), jnp.int32),
            pltpu.VMEM((2, CH), jnp.int32),
            pltpu.VMEM((2, CH), jnp.int32),
            pltpu.VMEM((2, 2, CH), jnp.int32),
            pltpu.VMEM((2, CH), jnp.int32),
            pltpu.VMEM((2, CH, D), jnp.float32),
            pltpu.SemaphoreType.DMA,
            pltpu.SemaphoreType.DMA,
            pltpu.SemaphoreType.DMA,
            pltpu.SemaphoreType.DMA,
            pltpu.SemaphoreType.DMA,
            pltpu.SemaphoreType.DMA,
            pltpu.SemaphoreType.DMA,
            pltpu.SemaphoreType.DMA,
        ],
        compiler_params=pltpu.CompilerParams(needs_layout_passes=False),
    )


def _sc_gather(table, row, col, nt):
    return _sc_gather_kernel(row.shape[0])(table, row, col, nt)


def _edge_body(g_ref, ea_ref, p_ref, w1a_ref, w2_ref, b2_ref, o_ref):
    gb = g_ref[...]
    ea = ea_ref[...]
    pb = p_ref[...]
    colj = lax.broadcasted_iota(jnp.int32, (BE, 4 * DE), 1) // DE
    ea4 = jnp.concatenate([ea, ea, ea, ea], axis=1)
    a64 = jnp.where(colj == pb, ea4, 0.0)
    pre = gb + jnp.dot(a64, w1a_ref[...], preferred_element_type=jnp.float32)
    h1 = jnp.maximum(pre, 0.0)
    acc = jnp.zeros((BE, H), jnp.float32)
    b2s = jnp.zeros((BE, H), jnp.float32)
    for j in range(4):
        hj = jnp.where(pb == j, h1, 0.0)
        acc = acc + jnp.dot(hj, w2_ref[j], preferred_element_type=jnp.float32)
        b2s = b2s + jnp.where(pb == j, b2_ref[j], 0.0)
    o_ref[...] = jnp.maximum(acc + b2s, 0.0)


def _edge_mlp(g, ea, p2, w1a, w2s, b2s):
    n = g.shape[0]
    return pl.pallas_call(
        _edge_body,
        grid=(n // BE,),
        in_specs=[
            pl.BlockSpec((BE, H), lambda i: (i, 0)),
            pl.BlockSpec((BE, DE), lambda i: (i, 0)),
            pl.BlockSpec((BE, 1), lambda i: (i, 0)),
            pl.BlockSpec((4 * DE, H), lambda i: (0, 0)),
            pl.BlockSpec((4, H, H), lambda i: (0, 0, 0)),
            pl.BlockSpec((4, H), lambda i: (0, 0)),
        ],
        out_specs=pl.BlockSpec((BE, H), lambda i: (i, 0)),
        out_shape=jax.ShapeDtypeStruct((n, H), jnp.float32),
    )(g, ea, p2, w1a, w2s, b2s)


def _scatter_body(nchunk, eo_hbm, row_hbm, z_hbm, agg_hbm, shared, vbuf, idxbuf,
                  si0, si1, ss0, ss1):
    kmax = -(-nchunk // NW)
    c = lax.axis_index("c")
    s = lax.axis_index("s")
    r0 = 632
    lo = s * r0

    @pl.when(s < NS - 1)
    def _():
        pltpu.sync_copy(z_hbm.at[c, pl.ds(lo, r0)], shared.at[pl.ds(lo, r0)])

    @pl.when(s == NS - 1)
    def _():
        pltpu.sync_copy(z_hbm.at[c, pl.ds(lo, N - (NS - 1) * r0)],
                        shared.at[pl.ds(lo, N - (NS - 1) * r0)])

    plsc.subcore_barrier()

    sin = (si0, si1)
    ssc = (ss0, ss1)

    def cid_of(j):
        return c + NC * s + NW * j

    def valid(j):
        return (j >= 0) & (cid_of(j) < nchunk)

    def start_in(j, slot):
        @pl.when(valid(j))
        def _():
            eb = cid_of(j) * CH
            pltpu.async_copy(row_hbm.at[pl.ds(eb, CH)], idxbuf.at[slot], sin[slot])
            pltpu.async_copy(eo_hbm.at[pl.ds(eb, CH)], vbuf.at[slot], sin[slot])

    def step(j, slot):
        oslot = 1 - slot

        @pl.when(valid(j - 1))
        def _():
            pltpu.make_async_copy(vbuf.at[oslot], shared.at[idxbuf.at[oslot]], ssc[oslot]).wait()

        start_in(j + 1, oslot)

        @pl.when(valid(j))
        def _():
            eb = cid_of(j) * CH
            pltpu.make_async_copy(row_hbm.at[pl.ds(eb, CH)], idxbuf.at[slot], sin[slot]).wait()
            pltpu.make_async_copy(eo_hbm.at[pl.ds(eb, CH)], vbuf.at[slot], sin[slot]).wait()
            pltpu.async_copy(vbuf.at[slot], shared.at[idxbuf.at[slot]], ssc[slot], add=True)

    start_in(0, 0)
    nit = kmax + 2 + (kmax % 2)

    def two(k2, carry):
        step(2 * k2, 0)
        step(2 * k2 + 1, 1)
        return carry

    lax.fori_loop(0, nit // 2, two, 0)
    plsc.subcore_barrier()

    @pl.when(s < NS - 1)
    def _():
        pltpu.sync_copy(shared.at[pl.ds(lo, r0)], agg_hbm.at[c, pl.ds(lo, r0)])

    @pl.when(s == NS - 1)
    def _():
        pltpu.sync_copy(shared.at[pl.ds(lo, N - (NS - 1) * r0)],
                        agg_hbm.at[c, pl.ds(lo, N - (NS - 1) * r0)])


@functools.cache
def _sc_scatter_kernel(n_edges):
    return pl.kernel(
        functools.partial(_scatter_body, n_edges // CH),
        out_type=jax.ShapeDtypeStruct((2, N, D), jnp.float32),
        mesh=_mesh(),
        scratch_types=[
            pltpu.VMEM_SHARED((N, D), jnp.float32),
            pltpu.VMEM((2, CH, D), jnp.float32),
            pltpu.VMEM((2, CH), jnp.int32),
            pltpu.SemaphoreType.DMA,
            pltpu.SemaphoreType.DMA,
            pltpu.SemaphoreType.DMA,
            pltpu.SemaphoreType.DMA,
        ],
        compiler_params=pltpu.CompilerParams(needs_layout_passes=False),
    )


def _sc_scatter(e_out, row, zeros):
    return _sc_scatter_kernel(row.shape[0])(e_out, row, zeros)


def _node_body(x_ref, a_ref, nt_ref, w1_ref, b1_ref, w2_ref, b2_ref, o_ref):
    xb = x_ref[...]
    ag = a_ref[0] + a_ref[1]
    nt = nt_ref[...]
    outs = []
    for t in range(2):
        h1 = jnp.maximum(
            jnp.dot(xb, w1_ref[t, :D], preferred_element_type=jnp.float32)
            + jnp.dot(ag, w1_ref[t, D:], preferred_element_type=jnp.float32)
            + b1_ref[t], 0.0)
        outs.append(jnp.dot(h1, w2_ref[t], preferred_element_type=jnp.float32) + b2_ref[t])
    o_ref[...] = jnp.where(nt == 0, outs[0], outs[1]) + xb


def _node_mlp(x0, aggp, nt2, w1, b1, w2, b2):
    return pl.pallas_call(
        _node_body,
        grid=(NB_NODE,),
        in_specs=[
            pl.BlockSpec((BN, D), lambda i: (i, 0)),
            pl.BlockSpec((2, BN, D), lambda i: (0, i, 0)),
            pl.BlockSpec((BN, 1), lambda i: (i, 0)),
            pl.BlockSpec((2, 2 * D, H), lambda i: (0, 0, 0)),
            pl.BlockSpec((2, H), lambda i: (0, 0)),
            pl.BlockSpec((2, H, H), lambda i: (0, 0, 0)),
            pl.BlockSpec((2, H), lambda i: (0, 0)),
        ],
        out_specs=pl.BlockSpec((BN, H), lambda i: (i, 0)),
        out_shape=jax.ShapeDtypeStruct((N, H), jnp.float32),
    )(x0, aggp, nt2, w1, b1, w2, b2)


def kernel(x, edge_attr, params, edge_index, node_type):
    x0 = x[:, 0]
    row = edge_index[0]
    col = edge_index[1]
    nt = node_type.astype(jnp.int32)
    nt2 = nt[:, None]

    w1 = {(a, b): params["edge_%d_%d_W1" % (a, b)] for a in (0, 1) for b in (0, 1)}
    b1 = {(a, b): params["edge_%d_%d_b1" % (a, b)] for a in (0, 1) for b in (0, 1)}
    wstk = jnp.stack([
        jnp.stack([jnp.stack([w1[(own, other)][:D] for other in (0, 1)])
                   for own in (0, 1)]),
        jnp.stack([jnp.stack([w1[(other, own)][D:2 * D] for other in (0, 1)])
                   for own in (0, 1)]),
    ])
    bstk = jnp.stack([
        jnp.stack([jnp.stack([b1[(own, other)] for other in (0, 1)])
                   for own in (0, 1)]),
        jnp.zeros((2, 2, H), jnp.float32),
    ])

    w1a = jnp.concatenate([w1[(j // 2, j % 2)][2 * D:] for j in range(4)], axis=0)
    w2s = jnp.stack([params["edge_%d_%d_W2" % (j // 2, j % 2)] for j in range(4)])
    b2s = jnp.stack([params["edge_%d_%d_b2" % (j // 2, j % 2)] for j in range(4)])

    wn1 = jnp.stack([params["node_%d_W1" % t] for t in (0, 1)])
    bn1 = jnp.stack([params["node_%d_b1" % t] for t in (0, 1)])
    wn2 = jnp.stack([params["node_%d_W2" % t] for t in (0, 1)])
    bn2 = jnp.stack([params["node_%d_b2" % t] for t in (0, 1)])

    table = _build_table(x0, nt2, wstk, bstk).reshape(4 * N, D)
    e2 = E // 2
    agg = jnp.zeros((2, N, D), jnp.float32)
    eo_halves = []
    for lo in (0, e2):
        rr = lax.slice(row, (lo,), (lo + e2,))
        cc = lax.slice(col, (lo,), (lo + e2,))
        ea = lax.slice(edge_attr, (lo, 0), (lo + e2, DE))
        g2, p = _sc_gather(table, rr, cc, nt)
        eo = _edge_mlp(g2, ea, p[:, None], w1a, w2s, b2s)
        agg = _sc_scatter(eo, rr, agg)
        eo_halves.append(eo)
    e_out = jnp.concatenate(eo_halves, axis=0)
    h = _node_mlp(x0, agg, nt2, wn1, bn1, wn2, bn2)
    return (h[:, None, :], edge_index, e_out, node_type)


# speedup vs baseline: 1.1120x; 1.1008x over previous
import functools

import jax
import jax.numpy as jnp
from jax import lax
from jax.experimental import pallas as pl
from jax.experimental.pallas import tpu as pltpu
from jax.experimental.pallas import tpu_sc as plsc

N = 10000
E = 160000
D = 128
DE = 16
H = 128
TWO_N = 2 * N

NC = 2
NS = 16
NW = NC * NS

BN = 1000
NB_NODE = N // BN
BE = 2000
NB_EDGE = E // BE
CH = 128
NCHUNK = E // CH
KMAX = -(-NCHUNK // NW)

@functools.cache
def _mesh():
    return plsc.VectorSubcoreMesh(core_axis_name="c", subcore_axis_name="s",
                                  num_cores=NC, num_subcores=NS)


def _table_body(x_ref, nt_ref, w_ref, b_ref, o_ref):
    xb = x_ref[...]
    nt = nt_ref[...]
    p0 = jnp.dot(xb, w_ref[0, 0], preferred_element_type=jnp.float32) + b_ref[0, 0, 0]
    p1 = jnp.dot(xb, w_ref[0, 1], preferred_element_type=jnp.float32) + b_ref[0, 1, 0]
    o_ref[...] = jnp.where(nt == 0, p0, p1)


def _build_table(x0, nt2, wstk4, bstk4):
    return pl.pallas_call(
        _table_body,
        grid=(NB_NODE, 4),
        in_specs=[
            pl.BlockSpec((BN, D), lambda nb, ho: (nb, 0)),
            pl.BlockSpec((BN, 1), lambda nb, ho: (nb, 0)),
            pl.BlockSpec((1, 2, D, D), lambda nb, ho: (ho, 0, 0, 0)),
            pl.BlockSpec((1, 2, 1, D), lambda nb, ho: (ho, 0, 0, 0)),
        ],
        out_specs=pl.BlockSpec((BN, D), lambda nb, ho: (ho * NB_NODE + nb, 0)),
        out_shape=jax.ShapeDtypeStruct((4 * N, D), jnp.float32),
    )(x0, nt2, wstk4, bstk4[:, :, None, :])


def _gather_body(nchunk, t_hbm, row_hbm, col_hbm, nt_hbm, g_hbm, p_hbm,
                 ntbuf, rbuf, cbuf, idxbuf, pbuf, dstbuf,
                 si0, si1, sg10, sg11, sg20, sg21, so0, so1):
    kmax = -(-nchunk // NW)
    c = lax.axis_index("c")
    s = lax.axis_index("s")
    w = s * NC + c
    pltpu.sync_copy(nt_hbm, ntbuf)
    sin = (si0, si1)
    sg1 = (sg10, sg11)
    sg2 = (sg20, sg21)
    sout = (so0, so1)

    def cid_of(j):
        return w + NW * j

    def valid(j):
        return (j >= 0) & (cid_of(j) < nchunk)

    def start_in(j, slot):
        @pl.when(valid(j))
        def _():
            eb = cid_of(j) * CH
            pltpu.async_copy(row_hbm.at[pl.ds(eb, CH)], rbuf.at[slot], sin[slot])
            pltpu.async_copy(col_hbm.at[pl.ds(eb, CH)], cbuf.at[slot], sin[slot])

    def step(j, slot):
        oslot = 1 - slot
        start_in(j + 1, oslot)

        @pl.when(valid(j - 2))
        def _():
            eb2 = cid_of(j - 2) * CH
            pltpu.make_async_copy(dstbuf.at[slot], g_hbm.at[pl.ds(eb2, CH)], sout[slot]).wait()
            pltpu.make_async_copy(pbuf.at[slot], p_hbm.at[pl.ds(eb2, CH)], sout[slot]).wait()

        @pl.when(valid(j))
        def _():
            eb = cid_of(j) * CH
            pltpu.make_async_copy(row_hbm.at[pl.ds(eb, CH)], rbuf.at[slot], sin[slot]).wait()
            pltpu.make_async_copy(col_hbm.at[pl.ds(eb, CH)], cbuf.at[slot], sin[slot]).wait()
            for g in range(8):
                sl = pl.ds(g * 16, 16)
                r16 = rbuf[slot, sl]
                c16 = cbuf[slot, sl]
                tr = plsc.load_gather(ntbuf, [r16])
                tc = plsc.load_gather(ntbuf, [c16])
                idxbuf[slot, 0, sl] = tc * N + r16
                idxbuf[slot, 1, sl] = TWO_N + tr * N + c16
                pbuf[slot, sl] = 2 * tr + tc

        @pl.when(valid(j))
        def _():
            pltpu.async_copy(t_hbm.at[idxbuf.at[slot, 0]], dstbuf.at[slot], sg1[slot])

        @pl.when(valid(j - 1))
        def _():
            eb1 = cid_of(j - 1) * CH
            pltpu.make_async_copy(t_hbm.at[idxbuf.at[oslot, 1]], dstbuf.at[oslot], sg2[oslot]).wait()
            pltpu.async_copy(dstbuf.at[oslot], g_hbm.at[pl.ds(eb1, CH)], sout[oslot])
            pltpu.async_copy(pbuf.at[oslot], p_hbm.at[pl.ds(eb1, CH)], sout[oslot])

        @pl.when(valid(j))
        def _():
            pltpu.make_async_copy(t_hbm.at[idxbuf.at[slot, 0]], dstbuf.at[slot], sg1[slot]).wait()
            pltpu.async_copy(t_hbm.at[idxbuf.at[slot, 1]], dstbuf.at[slot], sg2[slot], add=True)

    start_in(0, 0)
    nit = kmax + 2 + (kmax % 2)

    def two(k2, carry):
        step(2 * k2, 0)
        step(2 * k2 + 1, 1)
        return carry

    lax.fori_loop(0, nit // 2, two, 0)


@functools.cache
def _sc_gather_kernel(n_edges):
    return pl.kernel(
        functools.partial(_gather_body, n_edges // CH),
        out_type=(
            jax.ShapeDtypeStruct((n_edges, D), jnp.float32),
            jax.ShapeDtypeStruct((n_edges,), jnp.int32),
        ),
        mesh=_mesh(),
        scratch_types=[
            pltpu.VMEM((N,), jnp.int32),
            pltpu.VMEM((2, CH), jnp.int32),
            pltpu.VMEM((2, CH), jnp.int32),
            pltpu.VMEM((2, 2, CH), jnp.int32),
            pltpu.VMEM((2, CH), jnp.int32),
            pltpu.VMEM((2, CH, D), jnp.float32),
            pltpu.SemaphoreType.DMA,
            pltpu.SemaphoreType.DMA,
            pltpu.SemaphoreType.DMA,
            pltpu.SemaphoreType.DMA,
            pltpu.SemaphoreType.DMA,
            pltpu.SemaphoreType.DMA,
            pltpu.SemaphoreType.DMA,
            pltpu.SemaphoreType.DMA,
        ],
        compiler_params=pltpu.CompilerParams(needs_layout_passes=False),
    )


def _sc_gather(table, row, col, nt):
    return _sc_gather_kernel(row.shape[0])(table, row, col, nt)


def _edge_body(g_ref, ea_ref, p_ref, w1a_ref, w2_ref, b2_ref, o_ref):
    gb = g_ref[...]
    ea = ea_ref[...]
    pb = p_ref[...]
    colj = lax.broadcasted_iota(jnp.int32, (BE, 4 * DE), 1) // DE
    ea4 = jnp.concatenate([ea, ea, ea, ea], axis=1)
    a64 = jnp.where(colj == pb, ea4, 0.0)
    pre = gb + jnp.dot(a64, w1a_ref[...], preferred_element_type=jnp.float32)
    h1 = jnp.maximum(pre, 0.0)
    acc = jnp.zeros((BE, H), jnp.float32)
    b2s = jnp.zeros((BE, H), jnp.float32)
    for j in range(4):
        hj = jnp.where(pb == j, h1, 0.0)
        acc = acc + jnp.dot(hj, w2_ref[j], preferred_element_type=jnp.float32)
        b2s = b2s + jnp.where(pb == j, b2_ref[j], 0.0)
    o_ref[...] = jnp.maximum(acc + b2s, 0.0)


def _edge_mlp(g, ea, p2, w1a, w2s, b2s):
    n = g.shape[0]
    return pl.pallas_call(
        _edge_body,
        grid=(n // BE,),
        in_specs=[
            pl.BlockSpec((BE, H), lambda i: (i, 0)),
            pl.BlockSpec((BE, DE), lambda i: (i, 0)),
            pl.BlockSpec((BE, 1), lambda i: (i, 0)),
            pl.BlockSpec((4 * DE, H), lambda i: (0, 0)),
            pl.BlockSpec((4, H, H), lambda i: (0, 0, 0)),
            pl.BlockSpec((4, H), lambda i: (0, 0)),
        ],
        out_specs=pl.BlockSpec((BE, H), lambda i: (i, 0)),
        out_shape=jax.ShapeDtypeStruct((n, H), jnp.float32),
    )(g, ea, p2, w1a, w2s, b2s)


def _scatter_body(nchunk, eo_hbm, row_hbm, z_hbm, agg_hbm, shared, vbuf, idxbuf,
                  si0, si1, ss0, ss1):
    kmax = -(-nchunk // NW)
    c = lax.axis_index("c")
    s = lax.axis_index("s")
    r0 = 632
    lo = s * r0

    @pl.when(s < NS - 1)
    def _():
        pltpu.sync_copy(z_hbm.at[c, pl.ds(lo, r0)], shared.at[pl.ds(lo, r0)])

    @pl.when(s == NS - 1)
    def _():
        pltpu.sync_copy(z_hbm.at[c, pl.ds(lo, N - (NS - 1) * r0)],
                        shared.at[pl.ds(lo, N - (NS - 1) * r0)])

    plsc.subcore_barrier()

    sin = (si0, si1)
    ssc = (ss0, ss1)

    def cid_of(j):
        return c + NC * s + NW * j

    def valid(j):
        return (j >= 0) & (cid_of(j) < nchunk)

    def start_in(j, slot):
        @pl.when(valid(j))
        def _():
            eb = cid_of(j) * CH
            pltpu.async_copy(row_hbm.at[pl.ds(eb, CH)], idxbuf.at[slot], sin[slot])
            pltpu.async_copy(eo_hbm.at[pl.ds(eb, CH)], vbuf.at[slot], sin[slot])

    def step(j, slot):
        oslot = 1 - slot

        @pl.when(valid(j - 1))
        def _():
            pltpu.make_async_copy(vbuf.at[oslot], shared.at[idxbuf.at[oslot]], ssc[oslot]).wait()

        start_in(j + 1, oslot)

        @pl.when(valid(j))
        def _():
            eb = cid_of(j) * CH
            pltpu.make_async_copy(row_hbm.at[pl.ds(eb, CH)], idxbuf.at[slot], sin[slot]).wait()
            pltpu.make_async_copy(eo_hbm.at[pl.ds(eb, CH)], vbuf.at[slot], sin[slot]).wait()
            pltpu.async_copy(vbuf.at[slot], shared.at[idxbuf.at[slot]], ssc[slot], add=True)

    start_in(0, 0)
    nit = kmax + 2 + (kmax % 2)

    def two(k2, carry):
        step(2 * k2, 0)
        step(2 * k2 + 1, 1)
        return carry

    lax.fori_loop(0, nit // 2, two, 0)
    plsc.subcore_barrier()

    @pl.when(s < NS - 1)
    def _():
        pltpu.sync_copy(shared.at[pl.ds(lo, r0)], agg_hbm.at[c, pl.ds(lo, r0)])

    @pl.when(s == NS - 1)
    def _():
        pltpu.sync_copy(shared.at[pl.ds(lo, N - (NS - 1) * r0)],
                        agg_hbm.at[c, pl.ds(lo, N - (NS - 1) * r0)])


@functools.cache
def _sc_scatter_kernel(n_edges):
    return pl.kernel(
        functools.partial(_scatter_body, n_edges // CH),
        out_type=jax.ShapeDtypeStruct((2, N, D), jnp.float32),
        mesh=_mesh(),
        scratch_types=[
            pltpu.VMEM_SHARED((N, D), jnp.float32),
            pltpu.VMEM((2, CH, D), jnp.float32),
            pltpu.VMEM((2, CH), jnp.int32),
            pltpu.SemaphoreType.DMA,
            pltpu.SemaphoreType.DMA,
            pltpu.SemaphoreType.DMA,
            pltpu.SemaphoreType.DMA,
        ],
        compiler_params=pltpu.CompilerParams(needs_layout_passes=False),
    )


def _sc_scatter(e_out, row, zeros):
    return _sc_scatter_kernel(row.shape[0])(e_out, row, zeros)


def _node_body(x_ref, a_ref, nt_ref, w1_ref, b1_ref, w2_ref, b2_ref, o_ref):
    xb = x_ref[...]
    ag = a_ref[0] + a_ref[1]
    nt = nt_ref[...]
    outs = []
    for t in range(2):
        h1 = jnp.maximum(
            jnp.dot(xb, w1_ref[t, :D], preferred_element_type=jnp.float32)
            + jnp.dot(ag, w1_ref[t, D:], preferred_element_type=jnp.float32)
            + b1_ref[t], 0.0)
        outs.append(jnp.dot(h1, w2_ref[t], preferred_element_type=jnp.float32) + b2_ref[t])
    o_ref[...] = jnp.where(nt == 0, outs[0], outs[1]) + xb


def _node_mlp(x0, aggp, nt2, w1, b1, w2, b2):
    return pl.pallas_call(
        _node_body,
        grid=(NB_NODE,),
        in_specs=[
            pl.BlockSpec((BN, D), lambda i: (i, 0)),
            pl.BlockSpec((2, BN, D), lambda i: (0, i, 0)),
            pl.BlockSpec((BN, 1), lambda i: (i, 0)),
            pl.BlockSpec((2, 2 * D, H), lambda i: (0, 0, 0)),
            pl.BlockSpec((2, H), lambda i: (0, 0)),
            pl.BlockSpec((2, H, H), lambda i: (0, 0, 0)),
            pl.BlockSpec((2, H), lambda i: (0, 0)),
        ],
        out_specs=pl.BlockSpec((BN, H), lambda i: (i, 0)),
        out_shape=jax.ShapeDtypeStruct((N, H), jnp.float32),
    )(x0, aggp, nt2, w1, b1, w2, b2)


def kernel(x, edge_attr, params, edge_index, node_type):
    x0 = x[:, 0]
    row = edge_index[0]
    col = edge_index[1]
    nt = node_type.astype(jnp.int32)
    nt2 = nt[:, None]

    w1 = {(a, b): params["edge_%d_%d_W1" % (a, b)] for a in (0, 1) for b in (0, 1)}
    b1 = {(a, b): params["edge_%d_%d_b1" % (a, b)] for a in (0, 1) for b in (0, 1)}
    wstk4 = jnp.stack([
        jnp.stack([w1[(own, ho % 2)][:D] if ho < 2 else w1[(ho % 2, own)][D:2 * D]
                   for own in (0, 1)])
        for ho in range(4)
    ])
    bstk4 = jnp.stack([
        jnp.stack([b1[(own, ho % 2)] if ho < 2 else jnp.zeros((H,), jnp.float32)
                   for own in (0, 1)])
        for ho in range(4)
    ])

    w1a = jnp.concatenate([w1[(j // 2, j % 2)][2 * D:] for j in range(4)], axis=0)
    w2s = jnp.stack([params["edge_%d_%d_W2" % (j // 2, j % 2)] for j in range(4)])
    b2s = jnp.stack([params["edge_%d_%d_b2" % (j // 2, j % 2)] for j in range(4)])

    wn1 = jnp.stack([params["node_%d_W1" % t] for t in (0, 1)])
    bn1 = jnp.stack([params["node_%d_b1" % t] for t in (0, 1)])
    wn2 = jnp.stack([params["node_%d_W2" % t] for t in (0, 1)])
    bn2 = jnp.stack([params["node_%d_b2" % t] for t in (0, 1)])

    table = _build_table(x0, nt2, wstk4, bstk4)
    e2 = E // 2
    agg = jnp.zeros((2, N, D), jnp.float32)
    eo_halves = []
    for lo in (0, e2):
        rr = lax.slice(row, (lo,), (lo + e2,))
        cc = lax.slice(col, (lo,), (lo + e2,))
        ea = lax.slice(edge_attr, (lo, 0), (lo + e2, DE))
        g2, p = _sc_gather(table, rr, cc, nt)
        eo = _edge_mlp(g2, ea, p[:, None], w1a, w2s, b2s)
        agg = _sc_scatter(eo, rr, agg)
        eo_halves.append(eo)
    e_out = jnp.concatenate(eo_halves, axis=0)
    h = _node_mlp(x0, agg, nt2, wn1, bn1, wn2, bn2)
    return (h[:, None, :], edge_index, e_out, node_type)


# speedup vs baseline: 1.1342x; 1.0199x over previous
import functools

import jax
import jax.numpy as jnp
from jax import lax
from jax.experimental import pallas as pl
from jax.experimental.pallas import tpu as pltpu
from jax.experimental.pallas import tpu_sc as plsc

N = 10000
E = 160000
D = 128
DE = 16
H = 128
TWO_N = 2 * N

NC = 2
NS = 16
NW = NC * NS

BN = 1000
NB_NODE = N // BN
BE = 2000
NB_EDGE = E // BE
CH = 128
NCHUNK = E // CH
KMAX = -(-NCHUNK // NW)

@functools.cache
def _mesh():
    return plsc.VectorSubcoreMesh(core_axis_name="c", subcore_axis_name="s",
                                  num_cores=NC, num_subcores=NS)


def _table_body(x_ref, nt_ref, w_ref, b_ref, o_ref):
    xb = x_ref[...]
    nt = nt_ref[...]
    p0 = jnp.dot(xb, w_ref[0, 0], preferred_element_type=jnp.float32) + b_ref[0, 0, 0]
    p1 = jnp.dot(xb, w_ref[0, 1], preferred_element_type=jnp.float32) + b_ref[0, 1, 0]
    o_ref[...] = jnp.where(nt == 0, p0, p1)


def _build_table(x0, nt2, wstk4, bstk4):
    return pl.pallas_call(
        _table_body,
        grid=(NB_NODE, 4),
        in_specs=[
            pl.BlockSpec((BN, D), lambda nb, ho: (nb, 0)),
            pl.BlockSpec((BN, 1), lambda nb, ho: (nb, 0)),
            pl.BlockSpec((1, 2, D, D), lambda nb, ho: (ho, 0, 0, 0)),
            pl.BlockSpec((1, 2, 1, D), lambda nb, ho: (ho, 0, 0, 0)),
        ],
        out_specs=pl.BlockSpec((BN, D), lambda nb, ho: (ho * NB_NODE + nb, 0)),
        out_shape=jax.ShapeDtypeStruct((4 * N, D), jnp.float32),
    )(x0, nt2, wstk4, bstk4[:, :, None, :])


def _gather_body(nchunk, t_hbm, row_hbm, col_hbm, nt_hbm, g_hbm, p_hbm,
                 ntbuf, rbuf, cbuf, idxbuf, pbuf, dstbuf,
                 si0, si1, sg10, sg11, sg20, sg21, so0, so1):
    kmax = -(-nchunk // NW)
    c = lax.axis_index("c")
    s = lax.axis_index("s")
    w = s * NC + c
    pltpu.sync_copy(nt_hbm, ntbuf)
    sin = (si0, si1)
    sg1 = (sg10, sg11)
    sg2 = (sg20, sg21)
    sout = (so0, so1)

    def cid_of(j):
        return w + NW * j

    def valid(j):
        return (j >= 0) & (cid_of(j) < nchunk)

    def start_in(j, slot):
        @pl.when(valid(j))
        def _():
            eb = cid_of(j) * CH
            pltpu.async_copy(row_hbm.at[pl.ds(eb, CH)], rbuf.at[slot], sin[slot])
            pltpu.async_copy(col_hbm.at[pl.ds(eb, CH)], cbuf.at[slot], sin[slot])

    def step(j, slot):
        oslot = 1 - slot
        start_in(j + 1, oslot)

        @pl.when(valid(j - 2))
        def _():
            eb2 = cid_of(j - 2) * CH
            pltpu.make_async_copy(dstbuf.at[slot], g_hbm.at[pl.ds(eb2, CH)], sout[slot]).wait()
            pltpu.make_async_copy(pbuf.at[slot], p_hbm.at[pl.ds(eb2, CH)], sout[slot]).wait()

        @pl.when(valid(j))
        def _():
            eb = cid_of(j) * CH
            pltpu.make_async_copy(row_hbm.at[pl.ds(eb, CH)], rbuf.at[slot], sin[slot]).wait()
            pltpu.make_async_copy(col_hbm.at[pl.ds(eb, CH)], cbuf.at[slot], sin[slot]).wait()
            for g in range(8):
                sl = pl.ds(g * 16, 16)
                r16 = rbuf[slot, sl]
                c16 = cbuf[slot, sl]
                tr = plsc.load_gather(ntbuf, [r16])
                tc = plsc.load_gather(ntbuf, [c16])
                idxbuf[slot, 0, sl] = tc * N + r16
                idxbuf[slot, 1, sl] = TWO_N + tr * N + c16
                pbuf[slot, sl] = 2 * tr + tc

        @pl.when(valid(j))
        def _():
            pltpu.async_copy(t_hbm.at[idxbuf.at[slot, 0]], dstbuf.at[slot], sg1[slot])

        @pl.when(valid(j - 1))
        def _():
            eb1 = cid_of(j - 1) * CH
            pltpu.make_async_copy(t_hbm.at[idxbuf.at[oslot, 1]], dstbuf.at[oslot], sg2[oslot]).wait()
            pltpu.async_copy(dstbuf.at[oslot], g_hbm.at[pl.ds(eb1, CH)], sout[oslot])
            pltpu.async_copy(pbuf.at[oslot], p_hbm.at[pl.ds(eb1, CH)], sout[oslot])

        @pl.when(valid(j))
        def _():
            pltpu.make_async_copy(t_hbm.at[idxbuf.at[slot, 0]], dstbuf.at[slot], sg1[slot]).wait()
            pltpu.async_copy(t_hbm.at[idxbuf.at[slot, 1]], dstbuf.at[slot], sg2[slot], add=True)

    start_in(0, 0)
    nit = kmax + 2 + (kmax % 2)

    def two(k2, carry):
        step(2 * k2, 0)
        step(2 * k2 + 1, 1)
        return carry

    lax.fori_loop(0, nit // 2, two, 0)


@functools.cache
def _sc_gather_kernel(n_edges):
    return pl.kernel(
        functools.partial(_gather_body, n_edges // CH),
        out_type=(
            jax.ShapeDtypeStruct((n_edges, D), jnp.float32),
            jax.ShapeDtypeStruct((n_edges,), jnp.int32),
        ),
        mesh=_mesh(),
        scratch_types=[
            pltpu.VMEM((N,), jnp.int32),
            pltpu.VMEM((2, CH), jnp.int32),
            pltpu.VMEM((2, CH), jnp.int32),
            pltpu.VMEM((2, 2, CH), jnp.int32),
            pltpu.VMEM((2, CH), jnp.int32),
            pltpu.VMEM((2, CH, D), jnp.float32),
            pltpu.SemaphoreType.DMA,
            pltpu.SemaphoreType.DMA,
            pltpu.SemaphoreType.DMA,
            pltpu.SemaphoreType.DMA,
            pltpu.SemaphoreType.DMA,
            pltpu.SemaphoreType.DMA,
            pltpu.SemaphoreType.DMA,
            pltpu.SemaphoreType.DMA,
        ],
        compiler_params=pltpu.CompilerParams(needs_layout_passes=False),
    )


def _sc_gather(table, row, col, nt):
    return _sc_gather_kernel(row.shape[0])(table, row, col, nt)


def _edge_body(g_ref, ea_ref, p_ref, w1a_ref, w2_ref, b2_ref, o_ref):
    gb = g_ref[...]
    ea = ea_ref[...]
    pb = p_ref[...]
    colj = lax.broadcasted_iota(jnp.int32, (BE, 4 * DE), 1) // DE
    ea4 = jnp.concatenate([ea, ea, ea, ea], axis=1)
    a64 = jnp.where(colj == pb, ea4, 0.0)
    pre = gb + jnp.dot(a64, w1a_ref[...], preferred_element_type=jnp.float32)
    h1 = jnp.maximum(pre, 0.0)
    acc = jnp.zeros((BE, H), jnp.float32)
    b2s = jnp.zeros((BE, H), jnp.float32)
    for j in range(4):
        hj = jnp.where(pb == j, h1, 0.0)
        acc = acc + jnp.dot(hj, w2_ref[j], preferred_element_type=jnp.float32)
        b2s = b2s + jnp.where(pb == j, b2_ref[j], 0.0)
    o_ref[...] = jnp.maximum(acc + b2s, 0.0)


def _edge_mlp(g, ea_full, p2, w1a, w2s, b2s, blk_off):
    n = g.shape[0]
    return pl.pallas_call(
        _edge_body,
        grid=(n // BE,),
        in_specs=[
            pl.BlockSpec((BE, H), lambda i: (i, 0)),
            pl.BlockSpec((BE, DE), lambda i: (i + blk_off, 0)),
            pl.BlockSpec((BE, 1), lambda i: (i, 0)),
            pl.BlockSpec((4 * DE, H), lambda i: (0, 0)),
            pl.BlockSpec((4, H, H), lambda i: (0, 0, 0)),
            pl.BlockSpec((4, H), lambda i: (0, 0)),
        ],
        out_specs=pl.BlockSpec((BE, H), lambda i: (i, 0)),
        out_shape=jax.ShapeDtypeStruct((n, H), jnp.float32),
    )(g, ea_full, p2, w1a, w2s, b2s)


def _scatter_body(nchunk, eo_hbm, row_hbm, z_hbm, agg_hbm, shared, vbuf, idxbuf,
                  si0, si1, ss0, ss1):
    kmax = -(-nchunk // NW)
    c = lax.axis_index("c")
    s = lax.axis_index("s")
    r0 = 632
    lo = s * r0

    @pl.when(s < NS - 1)
    def _():
        pltpu.sync_copy(z_hbm.at[c, pl.ds(lo, r0)], shared.at[pl.ds(lo, r0)])

    @pl.when(s == NS - 1)
    def _():
        pltpu.sync_copy(z_hbm.at[c, pl.ds(lo, N - (NS - 1) * r0)],
                        shared.at[pl.ds(lo, N - (NS - 1) * r0)])

    plsc.subcore_barrier()

    sin = (si0, si1)
    ssc = (ss0, ss1)

    def cid_of(j):
        return c + NC * s + NW * j

    def valid(j):
        return (j >= 0) & (cid_of(j) < nchunk)

    def start_in(j, slot):
        @pl.when(valid(j))
        def _():
            eb = cid_of(j) * CH
            pltpu.async_copy(row_hbm.at[pl.ds(eb, CH)], idxbuf.at[slot], sin[slot])
            pltpu.async_copy(eo_hbm.at[pl.ds(eb, CH)], vbuf.at[slot], sin[slot])

    def step(j, slot):
        oslot = 1 - slot

        @pl.when(valid(j - 1))
        def _():
            pltpu.make_async_copy(vbuf.at[oslot], shared.at[idxbuf.at[oslot]], ssc[oslot]).wait()

        start_in(j + 1, oslot)

        @pl.when(valid(j))
        def _():
            eb = cid_of(j) * CH
            pltpu.make_async_copy(row_hbm.at[pl.ds(eb, CH)], idxbuf.at[slot], sin[slot]).wait()
            pltpu.make_async_copy(eo_hbm.at[pl.ds(eb, CH)], vbuf.at[slot], sin[slot]).wait()
            pltpu.async_copy(vbuf.at[slot], shared.at[idxbuf.at[slot]], ssc[slot], add=True)

    start_in(0, 0)
    nit = kmax + 2 + (kmax % 2)

    def two(k2, carry):
        step(2 * k2, 0)
        step(2 * k2 + 1, 1)
        return carry

    lax.fori_loop(0, nit // 2, two, 0)
    plsc.subcore_barrier()

    @pl.when(s < NS - 1)
    def _():
        pltpu.sync_copy(shared.at[pl.ds(lo, r0)], agg_hbm.at[c, pl.ds(lo, r0)])

    @pl.when(s == NS - 1)
    def _():
        pltpu.sync_copy(shared.at[pl.ds(lo, N - (NS - 1) * r0)],
                        agg_hbm.at[c, pl.ds(lo, N - (NS - 1) * r0)])


@functools.cache
def _sc_scatter_kernel(n_edges):
    return pl.kernel(
        functools.partial(_scatter_body, n_edges // CH),
        out_type=jax.ShapeDtypeStruct((2, N, D), jnp.float32),
        mesh=_mesh(),
        scratch_types=[
            pltpu.VMEM_SHARED((N, D), jnp.float32),
            pltpu.VMEM((2, CH, D), jnp.float32),
            pltpu.VMEM((2, CH), jnp.int32),
            pltpu.SemaphoreType.DMA,
            pltpu.SemaphoreType.DMA,
            pltpu.SemaphoreType.DMA,
            pltpu.SemaphoreType.DMA,
        ],
        compiler_params=pltpu.CompilerParams(needs_layout_passes=False),
    )


def _sc_scatter(e_out, row, zeros):
    return _sc_scatter_kernel(row.shape[0])(e_out, row, zeros)


def _node_body(x_ref, a_ref, nt_ref, w1_ref, b1_ref, w2_ref, b2_ref, o_ref):
    xb = x_ref[...]
    ag = a_ref[0] + a_ref[1]
    nt = nt_ref[...]
    outs = []
    for t in range(2):
        h1 = jnp.maximum(
            jnp.dot(xb, w1_ref[t, :D], preferred_element_type=jnp.float32)
            + jnp.dot(ag, w1_ref[t, D:], preferred_element_type=jnp.float32)
            + b1_ref[t], 0.0)
        outs.append(jnp.dot(h1, w2_ref[t], preferred_element_type=jnp.float32) + b2_ref[t])
    o_ref[...] = jnp.where(nt == 0, outs[0], outs[1]) + xb


def _node_mlp(x0, aggp, nt2, w1, b1, w2, b2):
    return pl.pallas_call(
        _node_body,
        grid=(NB_NODE,),
        in_specs=[
            pl.BlockSpec((BN, D), lambda i: (i, 0)),
            pl.BlockSpec((2, BN, D), lambda i: (0, i, 0)),
            pl.BlockSpec((BN, 1), lambda i: (i, 0)),
            pl.BlockSpec((2, 2 * D, H), lambda i: (0, 0, 0)),
            pl.BlockSpec((2, H), lambda i: (0, 0)),
            pl.BlockSpec((2, H, H), lambda i: (0, 0, 0)),
            pl.BlockSpec((2, H), lambda i: (0, 0)),
        ],
        out_specs=pl.BlockSpec((BN, H), lambda i: (i, 0)),
        out_shape=jax.ShapeDtypeStruct((N, H), jnp.float32),
    )(x0, aggp, nt2, w1, b1, w2, b2)


def kernel(x, edge_attr, params, edge_index, node_type):
    x0 = x[:, 0]
    row = edge_index[0]
    col = edge_index[1]
    nt = node_type.astype(jnp.int32)
    nt2 = nt[:, None]

    w1 = {(a, b): params["edge_%d_%d_W1" % (a, b)] for a in (0, 1) for b in (0, 1)}
    b1 = {(a, b): params["edge_%d_%d_b1" % (a, b)] for a in (0, 1) for b in (0, 1)}
    wstk4 = jnp.stack([
        jnp.stack([w1[(own, ho % 2)][:D] if ho < 2 else w1[(ho % 2, own)][D:2 * D]
                   for own in (0, 1)])
        for ho in range(4)
    ])
    bstk4 = jnp.stack([
        jnp.stack([b1[(own, ho % 2)] if ho < 2 else jnp.zeros((H,), jnp.float32)
                   for own in (0, 1)])
        for ho in range(4)
    ])

    w1a = jnp.concatenate([w1[(j // 2, j % 2)][2 * D:] for j in range(4)], axis=0)
    w2s = jnp.stack([params["edge_%d_%d_W2" % (j // 2, j % 2)] for j in range(4)])
    b2s = jnp.stack([params["edge_%d_%d_b2" % (j // 2, j % 2)] for j in range(4)])

    wn1 = jnp.stack([params["node_%d_W1" % t] for t in (0, 1)])
    bn1 = jnp.stack([params["node_%d_b1" % t] for t in (0, 1)])
    wn2 = jnp.stack([params["node_%d_W2" % t] for t in (0, 1)])
    bn2 = jnp.stack([params["node_%d_b2" % t] for t in (0, 1)])

    table = _build_table(x0, nt2, wstk4, bstk4)
    e2 = E // 2
    agg = jnp.zeros((2, N, D), jnp.float32)
    eo_halves = []
    for lo in (0, e2):
        rr = lax.slice(row, (lo,), (lo + e2,))
        cc = lax.slice(col, (lo,), (lo + e2,))
        g2, p = _sc_gather(table, rr, cc, nt)
        eo = _edge_mlp(g2, edge_attr, p[:, None], w1a, w2s, b2s, lo // BE)
        agg = _sc_scatter(eo, rr, agg)
        eo_halves.append(eo)
    e_out = jnp.concatenate(eo_halves, axis=0)
    h = _node_mlp(x0, agg, nt2, wn1, bn1, wn2, bn2)
    return (h[:, None, :], edge_index, e_out, node_type)
